# Initial kernel scaffold; baseline (speedup 1.0000x reference)
import jax
import jax.numpy as jnp
from jax.experimental import pallas as pl


def kernel(x, edge_index, W1, b1, Wih1, Whh1, bih1, bhh1, W2, b2, Wih2, Whh2, bih2, bhh2):
    raise NotImplementedError("write your pallas kernel here")


# speedup vs baseline: 10.4445x; 10.4445x over previous
import functools

import jax
import jax.numpy as jnp
from jax import lax
from jax.experimental import pallas as pl
from jax.experimental.pallas import tpu as pltpu
from jax.experimental.pallas import tpu_sc as plsc

N = 10000
E = 320000
D = 128

NP = 10240
NW = 32
CH = 128
EW = 10240
NCH = EW // CH
CPB = 16
NBLK = NCH // CPB
ROWS_PER_SUB = NP // 16


def _sc_segment_sum(xw, srcs, dsts):
  mesh = plsc.VectorSubcoreMesh(core_axis_name="c", subcore_axis_name="s")

  @functools.partial(
      pl.kernel,
      mesh=mesh,
      out_type=jax.ShapeDtypeStruct((2, NP, D), jnp.float32),
      scratch_types=[
          pltpu.VMEM((CPB, CH), jnp.int32),
          pltpu.VMEM((CPB, CH), jnp.int32),
          pltpu.VMEM((CH, D), jnp.float32),
          pltpu.VMEM((CH, D), jnp.float32),
          pltpu.VMEM_SHARED((NP, D), jnp.float32),
          pltpu.SemaphoreType.DMA,
          pltpu.SemaphoreType.DMA,
      ],
  )
  def seg_sum(xw_hbm, srcs_hbm, dsts_hbm, out_hbm,
              src_v, dst_v, rows_a, rows_b, acc_sh, sem_a, sem_b):
    cid = lax.axis_index("c")
    sid = lax.axis_index("s")
    wid = cid * 16 + sid

    def zero_rows(i, carry):
      rows_a[i // 8, pl.ds((i % 8) * 16, 16)] = jnp.zeros((16,), jnp.float32)
      return carry
    lax.fori_loop(0, (CH * D) // 16, zero_rows, 0)

    def zero_acc(j, carry):
      pltpu.sync_copy(rows_a, acc_sh.at[pl.ds(sid * ROWS_PER_SUB + j * CH, CH)])
      return carry
    lax.fori_loop(0, ROWS_PER_SUB // CH, zero_acc, 0)
    plsc.subcore_barrier()

    def block(b, carry):
      pltpu.sync_copy(srcs_hbm.at[wid, pl.ds(b * CPB, CPB)], src_v)
      pltpu.sync_copy(dsts_hbm.at[wid, pl.ds(b * CPB, CPB)], dst_v)

      pltpu.async_copy(xw_hbm.at[src_v.at[0]], rows_a, sem_a)

      def chunk(c, carry2):
        even = lax.rem(c, 2) == 0
        @pl.when(even)
        def _():
          @pl.when(c + 1 < CPB)
          def _():
            pltpu.async_copy(xw_hbm.at[src_v.at[c + 1]], rows_b, sem_b)
          pltpu.make_async_copy(xw_hbm.at[src_v.at[c]], rows_a, sem_a).wait()
          pltpu.sync_copy(rows_a, acc_sh.at[dst_v.at[c]], add=True)
        @pl.when(jnp.logical_not(even))
        def _():
          @pl.when(c + 1 < CPB)
          def _():
            pltpu.async_copy(xw_hbm.at[src_v.at[c + 1]], rows_a, sem_a)
          pltpu.make_async_copy(xw_hbm.at[src_v.at[c]], rows_b, sem_b).wait()
          pltpu.sync_copy(rows_b, acc_sh.at[dst_v.at[c]], add=True)
        return carry2
      lax.fori_loop(0, CPB, chunk, 0)
      return carry
    lax.fori_loop(0, NBLK, block, 0)
    plsc.subcore_barrier()

    pltpu.sync_copy(
        acc_sh.at[pl.ds(sid * ROWS_PER_SUB, ROWS_PER_SUB)],
        out_hbm.at[cid, pl.ds(sid * ROWS_PER_SUB, ROWS_PER_SUB)])

  return seg_sum(xw, srcs, dsts)


def _mm_bias_kernel(x_ref, w_ref, b_ref, o_ref):
  o_ref[...] = (
      jnp.dot(x_ref[...], w_ref[...], preferred_element_type=jnp.float32)
      + b_ref[...])


def _mm_bias(x, w, b):
  blk = 1024
  grid = NP // blk
  k = w.shape[1]
  return pl.pallas_call(
      _mm_bias_kernel,
      grid=(grid,),
      in_specs=[
          pl.BlockSpec((blk, D), lambda i: (i, 0)),
          pl.BlockSpec((D, k), lambda i: (0, 0)),
          pl.BlockSpec((1, k), lambda i: (0, 0)),
      ],
      out_specs=pl.BlockSpec((blk, k), lambda i: (i, 0)),
      out_shape=jax.ShapeDtypeStruct((NP, k), jnp.float32),
  )(x, w, b.reshape(1, k))


def _gru_kernel(do_next, m0_ref, m1_ref, x_ref, wih_ref, whh_ref, bih_ref,
                bhh_ref, wn_ref, bn_ref, h_ref, xwn_ref):
  m = m0_ref[...] + m1_ref[...]
  x = x_ref[...]
  gi = jnp.dot(m, wih_ref[...], preferred_element_type=jnp.float32) + bih_ref[...]
  gh = jnp.dot(x, whh_ref[...], preferred_element_type=jnp.float32) + bhh_ref[...]
  r = jax.nn.sigmoid(gi[:, :D] + gh[:, :D])
  z = jax.nn.sigmoid(gi[:, D:2 * D] + gh[:, D:2 * D])
  n = jnp.tanh(gi[:, 2 * D:] + r * gh[:, 2 * D:])
  h = jax.nn.relu((1.0 - z) * n + z * x)
  h_ref[...] = h
  if do_next:
    xwn_ref[...] = (
        jnp.dot(h, wn_ref[...], preferred_element_type=jnp.float32)
        + bn_ref[...])


def _gru(m0, m1, x, wih, whh, bih, bhh, wn=None, bn=None):
  do_next = wn is not None
  if wn is None:
    wn = jnp.zeros((D, D), jnp.float32)
    bn = jnp.zeros((D,), jnp.float32)
  blk = 1024
  grid = NP // blk
  full = lambda r, c: pl.BlockSpec((r, c), lambda i: (0, 0))
  row = lambda c: pl.BlockSpec((blk, c), lambda i: (i, 0))
  h, xwn = pl.pallas_call(
      functools.partial(_gru_kernel, do_next),
      grid=(grid,),
      in_specs=[
          row(D), row(D), row(D),
          full(D, 3 * D), full(D, 3 * D), full(1, 3 * D), full(1, 3 * D),
          full(D, D), full(1, D),
      ],
      out_specs=[row(D), row(D)],
      out_shape=[
          jax.ShapeDtypeStruct((NP, D), jnp.float32),
          jax.ShapeDtypeStruct((NP, D), jnp.float32),
      ],
  )(m0, m1, x, wih, whh, bih.reshape(1, -1), bhh.reshape(1, -1), wn,
    bn.reshape(1, -1))
  return (h, xwn) if do_next else (h, None)


def kernel(x, edge_index, W1, b1, Wih1, Whh1, bih1, bhh1,
           W2, b2, Wih2, Whh2, bih2, bhh2):
  x_pad = jnp.pad(x, ((0, NP - N), (0, 0)))
  src = edge_index[0]
  dst = edge_index[1]
  npad = NW * EW - E
  pad_iota = jnp.arange(npad, dtype=jnp.int32)
  src_p = jnp.concatenate([src, pad_iota % N])
  dst_p = jnp.concatenate([dst, N + pad_iota % (NP - N)])
  srcs = src_p.reshape(NW, NCH, CH)
  dsts = dst_p.reshape(NW, NCH, CH)

  xw1 = _mm_bias(x_pad, W1, b1)
  p1 = _sc_segment_sum(xw1, srcs, dsts)
  h1, xw2 = _gru(p1[0], p1[1], x_pad, Wih1, Whh1, bih1, bhh1, W2, b2)

  p2 = _sc_segment_sum(xw2, srcs, dsts)
  h2, _ = _gru(p2[0], p2[1], h1, Wih2, Whh2, bih2, bhh2)

  return h2[:N]


# speedup vs baseline: 11.3817x; 1.0897x over previous
import functools

import jax
import jax.numpy as jnp
from jax import lax
from jax.experimental import pallas as pl
from jax.experimental.pallas import tpu as pltpu
from jax.experimental.pallas import tpu_sc as plsc

N = 10000
E = 320000
D = 128

NP = 10240
NW = 32
CH = 128
EW = 10240
NCH = EW // CH
CPB = 16
NBLK = NCH // CPB
ROWS_PER_SUB = NP // 16
BLK = 2000


def _sc_segment_sum(xw, srcs, dsts, zrows):
  mesh = plsc.VectorSubcoreMesh(core_axis_name="c", subcore_axis_name="s")

  @functools.partial(
      pl.kernel,
      mesh=mesh,
      out_type=[
          jax.ShapeDtypeStruct((NP, D), jnp.float32),
          jax.ShapeDtypeStruct((NP, D), jnp.float32),
      ],
      scratch_types=[
          pltpu.VMEM((CPB, CH), jnp.int32),
          pltpu.VMEM((CPB, CH), jnp.int32),
          pltpu.VMEM((CH, D), jnp.float32),
          pltpu.VMEM((CH, D), jnp.float32),
          pltpu.VMEM_SHARED((NP, D), jnp.float32),
          pltpu.SemaphoreType.DMA,
          pltpu.SemaphoreType.DMA,
          pltpu.SemaphoreType.DMA,
          pltpu.SemaphoreType.DMA,
      ],
  )
  def seg_sum(xw_hbm, srcs_hbm, dsts_hbm, zrows_hbm, out0_hbm, out1_hbm,
              src_v, dst_v, rows_a, rows_b, acc_sh,
              gsem_a, gsem_b, ssem_a, ssem_b):
    cid = lax.axis_index("c")
    sid = lax.axis_index("s")
    wid = cid * 16 + sid
    bufs = (rows_a, rows_b)
    gsems = (gsem_a, gsem_b)
    ssems = (ssem_a, ssem_b)

    with jax.named_scope("acc_zero"):
      pltpu.sync_copy(zrows_hbm, rows_a)
      def zero_acc(j, carry):
        pltpu.sync_copy(rows_a, acc_sh.at[pl.ds(sid * ROWS_PER_SUB + j * CH, CH)])
        return carry
      lax.fori_loop(0, ROWS_PER_SUB // CH, zero_acc, 0)
      plsc.subcore_barrier()

    with jax.named_scope("edge_loop"):
      def block(b, carry):
        pltpu.sync_copy(srcs_hbm.at[wid, pl.ds(b * CPB, CPB)], src_v)
        pltpu.sync_copy(dsts_hbm.at[wid, pl.ds(b * CPB, CPB)], dst_v)

        gd = {}
        sd = {}
        for c in range(CPB):
          if c >= 2:
            sd[c - 2].wait()
          gd[c] = pltpu.async_copy(
              xw_hbm.at[src_v.at[c]], bufs[c % 2], gsems[c % 2])
          if c >= 1:
            gd[c - 1].wait()
            sd[c - 1] = pltpu.async_copy(
                bufs[(c - 1) % 2], acc_sh.at[dst_v.at[c - 1]],
                ssems[(c - 1) % 2], add=True)
        gd[CPB - 1].wait()
        sd[CPB - 1] = pltpu.async_copy(
            bufs[(CPB - 1) % 2], acc_sh.at[dst_v.at[CPB - 1]],
            ssems[(CPB - 1) % 2], add=True)
        sd[CPB - 2].wait()
        sd[CPB - 1].wait()
        return carry
      lax.fori_loop(0, NBLK, block, 0)
      plsc.subcore_barrier()

    with jax.named_scope("writeout"):
      @pl.when(cid == 0)
      def _():
        pltpu.sync_copy(
            acc_sh.at[pl.ds(sid * ROWS_PER_SUB, ROWS_PER_SUB)],
            out0_hbm.at[pl.ds(sid * ROWS_PER_SUB, ROWS_PER_SUB)])
      @pl.when(cid == 1)
      def _():
        pltpu.sync_copy(
            acc_sh.at[pl.ds(sid * ROWS_PER_SUB, ROWS_PER_SUB)],
            out1_hbm.at[pl.ds(sid * ROWS_PER_SUB, ROWS_PER_SUB)])

  return seg_sum(xw, srcs, dsts, zrows)


def _mm_bias_kernel(x_ref, w_ref, b_ref, o_ref):
  o_ref[...] = (
      jnp.dot(x_ref[...], w_ref[...], preferred_element_type=jnp.float32)
      + b_ref[...])


def _mm_bias(x, w, b):
  grid = N // BLK
  k = w.shape[1]
  return pl.pallas_call(
      _mm_bias_kernel,
      grid=(grid,),
      in_specs=[
          pl.BlockSpec((BLK, D), lambda i: (i, 0)),
          pl.BlockSpec((D, k), lambda i: (0, 0)),
          pl.BlockSpec((1, k), lambda i: (0, 0)),
      ],
      out_specs=pl.BlockSpec((BLK, k), lambda i: (i, 0)),
      out_shape=jax.ShapeDtypeStruct((N, k), jnp.float32),
  )(x, w, b.reshape(1, k))


def _gru_kernel(do_next, m0_ref, m1_ref, x_ref, wih_ref, whh_ref, bih_ref,
                bhh_ref, wn_ref, bn_ref, h_ref, xwn_ref):
  m = m0_ref[...] + m1_ref[...]
  x = x_ref[...]
  gi = jnp.dot(m, wih_ref[...], preferred_element_type=jnp.float32) + bih_ref[...]
  gh = jnp.dot(x, whh_ref[...], preferred_element_type=jnp.float32) + bhh_ref[...]
  r = jax.nn.sigmoid(gi[:, :D] + gh[:, :D])
  z = jax.nn.sigmoid(gi[:, D:2 * D] + gh[:, D:2 * D])
  n = jnp.tanh(gi[:, 2 * D:] + r * gh[:, 2 * D:])
  h = jax.nn.relu((1.0 - z) * n + z * x)
  h_ref[...] = h
  if do_next:
    xwn_ref[...] = (
        jnp.dot(h, wn_ref[...], preferred_element_type=jnp.float32)
        + bn_ref[...])


def _gru(m0, m1, x, wih, whh, bih, bhh, wn=None, bn=None):
  do_next = wn is not None
  if wn is None:
    wn = jnp.zeros((D, D), jnp.float32)
    bn = jnp.zeros((D,), jnp.float32)
  grid = N // BLK
  full = lambda r, c: pl.BlockSpec((r, c), lambda i: (0, 0))
  row = lambda c: pl.BlockSpec((BLK, c), lambda i: (i, 0))
  h, xwn = pl.pallas_call(
      functools.partial(_gru_kernel, do_next),
      grid=(grid,),
      in_specs=[
          row(D), row(D), row(D),
          full(D, 3 * D), full(D, 3 * D), full(1, 3 * D), full(1, 3 * D),
          full(D, D), full(1, D),
      ],
      out_specs=[row(D), row(D)],
      out_shape=[
          jax.ShapeDtypeStruct((N, D), jnp.float32),
          jax.ShapeDtypeStruct((N, D), jnp.float32),
      ],
  )(m0, m1, x, wih, whh, bih.reshape(1, -1), bhh.reshape(1, -1), wn,
    bn.reshape(1, -1))
  return (h, xwn) if do_next else (h, None)


def kernel(x, edge_index, W1, b1, Wih1, Whh1, bih1, bhh1,
           W2, b2, Wih2, Whh2, bih2, bhh2):
  src = edge_index[0]
  dst = edge_index[1]
  npad = NW * EW - E
  pad_iota = jnp.arange(npad, dtype=jnp.int32)
  src_p = jnp.concatenate([src, pad_iota % N])
  dst_p = jnp.concatenate([dst, N + pad_iota % (NP - N)])
  srcs = src_p.reshape(NW, NCH, CH)
  dsts = dst_p.reshape(NW, NCH, CH)
  zrows = jnp.zeros((CH, D), jnp.float32)

  xw1 = _mm_bias(x, W1, b1)
  p1a, p1b = _sc_segment_sum(xw1, srcs, dsts, zrows)
  h1, xw2 = _gru(p1a, p1b, x, Wih1, Whh1, bih1, bhh1, W2, b2)

  p2a, p2b = _sc_segment_sum(xw2, srcs, dsts, zrows)
  h2, _ = _gru(p2a, p2b, h1, Wih2, Whh2, bih2, bhh2)

  return h2
